# Initial kernel scaffold; baseline (speedup 1.0000x reference)
import jax
import jax.numpy as jnp
from jax.experimental import pallas as pl


def kernel(date, month_table, weekday_table, day_table):
    raise NotImplementedError("write your pallas kernel here")


# speedup vs baseline: 2.7405x; 2.7405x over previous
import jax
import jax.numpy as jnp
from jax import lax
from jax.experimental import pallas as pl
from jax.experimental.pallas import tpu as pltpu
from jax.experimental.pallas import tpu_sc as plsc

NC, NS, L = 2, 16, 16
NW = NC * NS
B, T, C = 16384, 200, 12
N = B * T
PER_W = N // NW
CHUNK = 2048
STEPS = PER_W // CHUNK
G = CHUNK // L


def _body(date_hbm, mt_hbm, wt_hbm, dt_hbm, out_hbm, mt_v, wt_v, dt_v,
          date_v, out_v):
    wid = lax.axis_index("s") * NC + lax.axis_index("c")
    pltpu.sync_copy(mt_hbm, mt_v)
    pltpu.sync_copy(wt_hbm, wt_v)
    pltpu.sync_copy(dt_hbm, dt_v)
    iota = jnp.arange(L, dtype=jnp.int32)
    base_el = wid * PER_W

    def step(j, carry):
        el0 = base_el + j * CHUNK
        pltpu.sync_copy(date_hbm.at[pl.ds(el0 * 3, CHUNK * 3)], date_v)

        def vec(g, c2):
            i0 = iota * 3 + g * (3 * L)
            d0 = plsc.load_gather(date_v, [i0])
            d1 = plsc.load_gather(date_v, [i0 + 1])
            d2 = plsc.load_gather(date_v, [i0 + 2])
            mb = d0 * 3
            wb = d1 * 3
            db = d2 * 6
            p = iota * 12 + g * (12 * L)
            for c in range(3):
                plsc.store_scatter(out_v, [p + c],
                                   plsc.load_gather(mt_v, [mb + c]))
            for c in range(3):
                plsc.store_scatter(out_v, [p + 3 + c],
                                   plsc.load_gather(wt_v, [wb + c]))
            for c in range(6):
                plsc.store_scatter(out_v, [p + 6 + c],
                                   plsc.load_gather(dt_v, [db + c]))
            return c2

        lax.fori_loop(0, G, vec, 0)
        pltpu.sync_copy(out_v, out_hbm.at[pl.ds(el0 * 12, CHUNK * 12)])
        return carry

    lax.fori_loop(0, STEPS, step, 0)


_sc_call = pl.kernel(
    _body,
    out_type=jax.ShapeDtypeStruct((N * 12,), jnp.float32),
    mesh=plsc.VectorSubcoreMesh(core_axis_name="c", subcore_axis_name="s"),
    compiler_params=pltpu.CompilerParams(needs_layout_passes=False),
    scratch_types=[
        pltpu.VMEM((48,), jnp.float32),
        pltpu.VMEM((32,), jnp.float32),
        pltpu.VMEM((192,), jnp.float32),
        pltpu.VMEM((CHUNK * 3,), jnp.int32),
        pltpu.VMEM((CHUNK * 12,), jnp.float32),
    ],
)


@jax.jit
def kernel(date, month_table, weekday_table, day_table):
    date_flat = date.astype(jnp.int32).reshape(-1)
    mt = jnp.pad(month_table.reshape(-1), (0, 48 - 39))
    wt = jnp.pad(weekday_table.reshape(-1), (0, 32 - 21))
    dt = day_table.reshape(-1)
    out = _sc_call(date_flat, mt, wt, dt)
    return out.reshape(B, T, C)


# speedup vs baseline: 101.3870x; 36.9954x over previous
import jax
import jax.numpy as jnp
from jax import lax
from jax.experimental import pallas as pl
from jax.experimental.pallas import tpu as pltpu
from jax.experimental.pallas import tpu_sc as plsc

NC, NS, L = 2, 16, 16
NW = NC * NS
B, T, C = 16384, 200, 12
BW = B // NW
RB = 8
NCHUNK = T // RB


def _body(date_hbm, mt_hbm, wt_hbm, dt_hbm, out_hbm, mt_v, wt_v, dt_v,
          in_v, out_v):
    wid = lax.axis_index("s") * NC + lax.axis_index("c")
    pltpu.sync_copy(mt_hbm, mt_v)
    pltpu.sync_copy(wt_hbm, wt_v)
    pltpu.sync_copy(dt_hbm, dt_v)
    b0 = wid * BW

    def chunk(j, carry):
        r0 = j * RB
        pltpu.sync_copy(date_hbm.at[:, pl.ds(r0, RB), pl.ds(b0, BW)], in_v)

        def row(r, c2):
            for k in range(BW // L):
                sl = pl.ds(k * L, L)
                d0 = in_v[0, r, sl]
                d1 = in_v[1, r, sl]
                d2 = in_v[2, r, sl]
                mb = d0 * 3
                wb = d1 * 3
                db = d2 * 6
                for c in range(3):
                    out_v[c, r, sl] = plsc.load_gather(mt_v, [mb + c])
                for c in range(3):
                    out_v[3 + c, r, sl] = plsc.load_gather(wt_v, [wb + c])
                for c in range(6):
                    out_v[6 + c, r, sl] = plsc.load_gather(dt_v, [db + c])
            return c2

        lax.fori_loop(0, RB, row, 0)
        pltpu.sync_copy(out_v, out_hbm.at[:, pl.ds(r0, RB), pl.ds(b0, BW)])
        return carry

    lax.fori_loop(0, NCHUNK, chunk, 0)


_sc_call = pl.kernel(
    _body,
    out_type=jax.ShapeDtypeStruct((C, T, B), jnp.float32),
    mesh=plsc.VectorSubcoreMesh(core_axis_name="c", subcore_axis_name="s"),
    compiler_params=pltpu.CompilerParams(needs_layout_passes=False),
    scratch_types=[
        pltpu.VMEM((48,), jnp.float32),
        pltpu.VMEM((32,), jnp.float32),
        pltpu.VMEM((192,), jnp.float32),
        pltpu.VMEM((3, RB, BW), jnp.int32),
        pltpu.VMEM((C, RB, BW), jnp.float32),
    ],
)


@jax.jit
def kernel(date, month_table, weekday_table, day_table):
    datep = jnp.transpose(date.astype(jnp.int32), (2, 1, 0))
    mt = jnp.pad(month_table.reshape(-1), (0, 48 - 39))
    wt = jnp.pad(weekday_table.reshape(-1), (0, 32 - 21))
    dt = day_table.reshape(-1)
    out = _sc_call(datep, mt, wt, dt)
    return jnp.transpose(out, (2, 1, 0))


# speedup vs baseline: 124.1715x; 1.2247x over previous
import jax
import jax.numpy as jnp
from jax import lax
from jax.experimental import pallas as pl
from jax.experimental.pallas import tpu as pltpu
from jax.experimental.pallas import tpu_sc as plsc

NC, NS, L = 2, 16, 16
NW = NC * NS
B, T, C = 16384, 200, 12
BW = B // NW
RB = 8
NCHUNK = T // RB


def _body(date_hbm, mt_hbm, wt_hbm, dt_hbm, out_hbm, mt_v, wt_v, dt_v,
          in_a, in_b, out_a, out_b, s_ia, s_ib, s_oa, s_ob):
    wid = lax.axis_index("s") * NC + lax.axis_index("c")
    pltpu.sync_copy(mt_hbm, mt_v)
    pltpu.sync_copy(wt_hbm, wt_v)
    pltpu.sync_copy(dt_hbm, dt_v)
    b0 = wid * BW

    def in_cp(j, buf, sem):
        return pltpu.make_async_copy(
            date_hbm.at[:, pl.ds(j * RB, RB), pl.ds(b0, BW)], buf, sem)

    def out_cp(j, buf, sem):
        return pltpu.make_async_copy(
            buf, out_hbm.at[:, pl.ds(j * RB, RB), pl.ds(b0, BW)], sem)

    def compute(in_v, out_v):
        def row(r, c2):
            for k in range(BW // L):
                sl = pl.ds(k * L, L)
                d0 = in_v[0, r, sl]
                d1 = in_v[1, r, sl]
                d2 = in_v[2, r, sl]
                mb = d0 * 3
                wb = d1 * 3
                db = d2 * 6
                for c in range(3):
                    out_v[c, r, sl] = plsc.load_gather(mt_v, [mb + c])
                for c in range(3):
                    out_v[3 + c, r, sl] = plsc.load_gather(wt_v, [wb + c])
                for c in range(6):
                    out_v[6 + c, r, sl] = plsc.load_gather(dt_v, [db + c])
            return c2

        lax.fori_loop(0, RB, row, 0)

    in_cp(0, in_a, s_ia).start()

    def iter2(jj, carry):
        j = 2 * jj
        in_cp(j, in_a, s_ia).wait()
        in_cp(j + 1, in_b, s_ib).start()

        @pl.when(jj > 0)
        def _():
            out_cp(j - 2, out_a, s_oa).wait()

        compute(in_a, out_a)
        out_cp(j, out_a, s_oa).start()

        in_cp(j + 1, in_b, s_ib).wait()
        in_cp(j + 2, in_a, s_ia).start()

        @pl.when(jj > 0)
        def _():
            out_cp(j - 1, out_b, s_ob).wait()

        compute(in_b, out_b)
        out_cp(j + 1, out_b, s_ob).start()
        return carry

    lax.fori_loop(0, (NCHUNK - 1) // 2, iter2, 0)

    last = NCHUNK - 1
    in_cp(last, in_a, s_ia).wait()
    out_cp(last - 2, out_a, s_oa).wait()
    compute(in_a, out_a)
    out_cp(last, out_a, s_oa).start()
    out_cp(last - 1, out_b, s_ob).wait()
    out_cp(last, out_a, s_oa).wait()


_sc_call = pl.kernel(
    _body,
    out_type=jax.ShapeDtypeStruct((C, T, B), jnp.float32),
    mesh=plsc.VectorSubcoreMesh(core_axis_name="c", subcore_axis_name="s"),
    compiler_params=pltpu.CompilerParams(needs_layout_passes=False),
    scratch_types=[
        pltpu.VMEM((48,), jnp.float32),
        pltpu.VMEM((32,), jnp.float32),
        pltpu.VMEM((192,), jnp.float32),
        pltpu.VMEM((3, RB, BW), jnp.int32),
        pltpu.VMEM((3, RB, BW), jnp.int32),
        pltpu.VMEM((C, RB, BW), jnp.float32),
        pltpu.VMEM((C, RB, BW), jnp.float32),
        pltpu.SemaphoreType.DMA,
        pltpu.SemaphoreType.DMA,
        pltpu.SemaphoreType.DMA,
        pltpu.SemaphoreType.DMA,
    ],
)


@jax.jit
def kernel(date, month_table, weekday_table, day_table):
    datep = jnp.transpose(date.astype(jnp.int32), (2, 1, 0))
    mt = jnp.pad(month_table.reshape(-1), (0, 48 - 39))
    wt = jnp.pad(weekday_table.reshape(-1), (0, 32 - 21))
    dt = day_table.reshape(-1)
    out = _sc_call(datep, mt, wt, dt)
    return jnp.transpose(out, (2, 1, 0))
